# baseline (speedup 1.0000x reference)
import functools

import jax
import jax.numpy as jnp
from jax import lax
from jax.experimental import pallas as pl
from jax.experimental.pallas import tpu as pltpu
from jax.experimental.pallas import tpu_sc as plsc

_TOTAL = 8192
_DIM = 64
_NW = 32
_B = _TOTAL // _NW
_L = 16


def _rsqrt(x):
    xi = plsc.bitcast(x, jnp.int32)
    yi = jnp.int32(0x5F3759DF) - (xi >> 1)
    y = plsc.bitcast(yi, jnp.float32)
    for _ in range(3):
        y = y * (1.5 - 0.5 * x * y * y)
    return y


@jax.jit
def _scores(h, r, t, ent2, rel_emb):
    @functools.partial(
        pl.kernel,
        mesh=plsc.VectorSubcoreMesh(core_axis_name="c", subcore_axis_name="s"),
        out_type=jax.ShapeDtypeStruct((_TOTAL,), jnp.float32),
        compiler_params=pltpu.CompilerParams(needs_layout_passes=False),
        scratch_types=[
            pltpu.VMEM((_B,), jnp.int32),
            pltpu.VMEM((_B,), jnp.int32),
            pltpu.VMEM((_B,), jnp.int32),
            pltpu.VMEM((_B,), jnp.int32),
            pltpu.VMEM((_B,), jnp.int32),
            pltpu.VMEM((_B,), jnp.int32),
            pltpu.VMEM((_B, 2 * _DIM), jnp.float32),
            pltpu.VMEM((_B, 2 * _DIM), jnp.float32),
            pltpu.VMEM((_B, 2 * _DIM), jnp.float32),
            pltpu.VMEM((_B,), jnp.float32),
            pltpu.SemaphoreType.DMA,
        ],
    )
    def k(h_hbm, r_hbm, t_hbm, ent_hbm, rel_hbm, out_hbm,
          hi, ri, ti, hi2, ri2, ti2, hrow, trow, rrow, sc, sem):
        wid = lax.axis_index("s") * 2 + lax.axis_index("c")
        base = wid * _B
        pltpu.sync_copy(h_hbm.at[pl.ds(base, _B)], hi)
        pltpu.sync_copy(r_hbm.at[pl.ds(base, _B)], ri)
        pltpu.sync_copy(t_hbm.at[pl.ds(base, _B)], ti)

        @pl.loop(0, _B // _L)
        def _shift(g):
            b = g * _L
            hi2[pl.ds(b, _L)] = hi[pl.ds(b, _L)] >> 1
            ri2[pl.ds(b, _L)] = ri[pl.ds(b, _L)] >> 1
            ti2[pl.ds(b, _L)] = ti[pl.ds(b, _L)] >> 1

        cph = pltpu.async_copy(ent_hbm.at[hi2], hrow, sem)
        cpr = pltpu.async_copy(rel_hbm.at[ri2], rrow, sem)
        cpt = pltpu.async_copy(ent_hbm.at[ti2], trow, sem)
        cph.wait()
        cpr.wait()
        cpt.wait()

        lanes = lax.iota(jnp.int32, _L)
        zero = jnp.zeros((_L,), jnp.float32)

        @pl.loop(0, _B // _L)
        def _grp(g):
            b = g * _L
            rows = b + lanes
            hoff = (hi[pl.ds(b, _L)] & 1) * _DIM
            roff = (ri[pl.ds(b, _L)] & 1) * _DIM
            toff = (ti[pl.ds(b, _L)] & 1) * _DIM
            vhh = vrr = vtt = vhr = vht = vrt = zero
            for c in range(_DIM):
                hc = plsc.load_gather(hrow, [rows, hoff + c])
                rc = plsc.load_gather(rrow, [rows, roff + c])
                tc = plsc.load_gather(trow, [rows, toff + c])
                vhh = vhh + hc * hc
                vrr = vrr + rc * rc
                vtt = vtt + tc * tc
                vhr = vhr + hc * rc
                vht = vht + hc * tc
                vrt = vrt + rc * tc
            s2 = 3.0 + 2.0 * (vhr * _rsqrt(vhh * vrr)
                              - vht * _rsqrt(vhh * vtt)
                              - vrt * _rsqrt(vrr * vtt))
            s2 = jnp.maximum(s2, 0.0)
            sc[pl.ds(b, _L)] = s2 * _rsqrt(jnp.maximum(s2, 1e-20))

        pltpu.sync_copy(sc, out_hbm.at[pl.ds(base, _B)])

    return k(h, r, t, ent2, rel_emb)


def kernel(h, r, t, ent_emb, rel_emb):
    h = h.astype(jnp.int32)
    r = r.astype(jnp.int32)
    t = t.astype(jnp.int32)
    ent2 = ent_emb.reshape(ent_emb.shape[0] // 2, 2 * _DIM)
    rel2 = rel_emb.reshape(rel_emb.shape[0] // 2, 2 * _DIM)
    score = _scores(h, r, t, ent2, rel2)
    bs = _TOTAL // 2
    p_score = score[:bs].reshape(1, bs).transpose(1, 0)
    n_score = score[bs:].reshape(1, bs).transpose(1, 0)
    return (p_score, n_score)
